# Initial kernel scaffold; baseline (speedup 1.0000x reference)
import jax
import jax.numpy as jnp
from jax.experimental import pallas as pl


def kernel(ContTensor, CatTensor, LabelTensor, DoseTensor, TimeDiffTensor, VTensor, VancoElTensor, PtList, LengList, embed_weight):
    raise NotImplementedError("write your pallas kernel here")


# speedup vs baseline: 1.8200x; 1.8200x over previous
import functools

import jax
import jax.numpy as jnp
from jax import lax
from jax.experimental import pallas as pl
from jax.experimental.pallas import tpu as pltpu
from jax.experimental.pallas import tpu_sc as plsc

B, T, NC, DC = 1024, 50, 26, 16
V, D = 1000000, 64
DOUT = D + DC
NPAIR = B * T
NWORKER = 32
PAIRS_PER_W = NPAIR // NWORKER
GROUP_PAIRS = 64
GROUPS = PAIRS_PER_W // GROUP_PAIRS
IDX_BLK = 104
BLKS_PER_GROUP = GROUP_PAIRS * NC // IDX_BLK
TOTAL_BLKS = NPAIR * NC // IDX_BLK
BLKS_PER_W = TOTAL_BLKS // NWORKER
LANES = 16


def _sc_body(idx_hbm, cont_hbm, table_hbm, out_hbm,
             idx_v, rows_v, cont_v, out_v, gsem):
    wid = lax.axis_index("s") * 2 + lax.axis_index("c")

    def group(g, carry):
        blk0 = wid * BLKS_PER_W + g * BLKS_PER_GROUP
        pair0 = wid * PAIRS_PER_W + g * GROUP_PAIRS

        pltpu.sync_copy(idx_hbm.at[pl.ds(blk0, BLKS_PER_GROUP)], idx_v)
        copies = []
        for j in range(BLKS_PER_GROUP):
            copies.append(pltpu.async_copy(
                table_hbm.at[idx_v.at[j]],
                rows_v.at[pl.ds(j * IDX_BLK, IDX_BLK)],
                gsem))
        pltpu.sync_copy(cont_hbm.at[pl.ds(pair0, GROUP_PAIRS)], cont_v)
        for cp in copies:
            cp.wait()

        def pair(p, c2):
            base = p * NC
            for c in range(D // LANES):
                acc = rows_v[base, pl.ds(c * LANES, LANES)]
                for r in range(1, NC):
                    acc = acc + rows_v[base + r, pl.ds(c * LANES, LANES)]
                out_v[p, pl.ds(c * LANES, LANES)] = acc
            out_v[p, pl.ds(D, LANES)] = cont_v[p, :]
            return c2

        lax.fori_loop(0, GROUP_PAIRS, pair, 0)
        pltpu.sync_copy(out_v, out_hbm.at[pl.ds(pair0, GROUP_PAIRS)])
        return carry

    lax.fori_loop(0, GROUPS, group, 0)


@jax.jit
def _emb_sum_concat(idx_blocks, cont2d, embed_weight):
    fn = pl.kernel(
        _sc_body,
        out_type=jax.ShapeDtypeStruct((NPAIR, DOUT), jnp.float32),
        mesh=plsc.VectorSubcoreMesh(core_axis_name="c", subcore_axis_name="s"),
        compiler_params=pltpu.CompilerParams(use_tc_tiling_on_sc=False),
        scratch_types=[
            pltpu.VMEM((BLKS_PER_GROUP, IDX_BLK), jnp.int32),
            pltpu.VMEM((GROUP_PAIRS * NC, D), jnp.float32),
            pltpu.VMEM((GROUP_PAIRS, DC), jnp.float32),
            pltpu.VMEM((GROUP_PAIRS, DOUT), jnp.float32),
            pltpu.SemaphoreType.DMA,
        ],
    )
    return fn(idx_blocks, cont2d, embed_weight)


def kernel(ContTensor, CatTensor, LabelTensor, DoseTensor, TimeDiffTensor,
           VTensor, VancoElTensor, PtList, LengList, embed_weight):
    idx_blocks = CatTensor.astype(jnp.int32).reshape(TOTAL_BLKS, IDX_BLK)
    cont2d = ContTensor.reshape(NPAIR, DC)
    out2d = _emb_sum_concat(idx_blocks, cont2d, embed_weight)
    outEmb = out2d.reshape(B, T, DOUT)
    return (outEmb, LabelTensor, LengList, DoseTensor, TimeDiffTensor,
            VTensor, VancoElTensor, PtList)


# speedup vs baseline: 2.0852x; 1.1458x over previous
import jax
import jax.numpy as jnp
from jax import lax
from jax.experimental import pallas as pl
from jax.experimental.pallas import tpu as pltpu
from jax.experimental.pallas import tpu_sc as plsc

B, T, NC, DC = 1024, 50, 26, 16
V, D = 1000000, 64
DOUT = D + DC
NPAIR = B * T
NWORKER = 32
PAIRS_PER_W = NPAIR // NWORKER
GROUP_PAIRS = 16
GROUPS = PAIRS_PER_W // GROUP_PAIRS
IDX_BLK = 104
BLKS_PER_GROUP = GROUP_PAIRS * NC // IDX_BLK
TOTAL_BLKS = NPAIR * NC // IDX_BLK
BLKS_PER_W = TOTAL_BLKS // NWORKER
LANES = 16
ROWS_PER_GROUP = GROUP_PAIRS * NC


def _sc_body(idx_hbm, cont_hbm, table_hbm, out_hbm,
             idx_v, cont_v, rows0, rows1, out0, out1,
             gsem0, gsem1, osem0, osem1):
    wid = lax.axis_index("s") * 2 + lax.axis_index("c")
    rows = (rows0, rows1)
    outs = (out0, out1)
    gsems = (gsem0, gsem1)
    osems = (osem0, osem1)

    pltpu.sync_copy(idx_hbm.at[pl.ds(wid * BLKS_PER_W, BLKS_PER_W)], idx_v)
    pltpu.sync_copy(cont_hbm.at[pl.ds(wid * PAIRS_PER_W, PAIRS_PER_W)], cont_v)

    def fire_gathers(g, buf):
        for j in range(BLKS_PER_GROUP):
            pltpu.async_copy(
                table_hbm.at[idx_v.at[g * BLKS_PER_GROUP + j]],
                rows[buf].at[pl.ds(j * IDX_BLK, IDX_BLK)],
                gsems[buf])

    def drain_gathers(g, buf):
        for j in range(BLKS_PER_GROUP):
            pltpu.make_async_copy(
                table_hbm.at[idx_v.at[g * BLKS_PER_GROUP + j]],
                rows[buf].at[pl.ds(j * IDX_BLK, IDX_BLK)],
                gsems[buf]).wait()

    fire_gathers(0, 0)

    def outer(g2, carry):
        for b in range(2):
            g = g2 * 2 + b
            nb = 1 - b

            @pl.when(g + 1 < GROUPS)
            def _():
                fire_gathers(g + 1, nb)

            drain_gathers(g, b)

            @pl.when(g >= 2)
            def _():
                pltpu.make_async_copy(
                    outs[b], out_hbm.at[pl.ds(0, GROUP_PAIRS)],
                    osems[b]).wait()

            def pair(p, c2):
                base = p * NC
                for c in range(D // LANES):
                    acc = rows[b][base, pl.ds(c * LANES, LANES)]
                    for r in range(1, NC):
                        acc = acc + rows[b][base + r, pl.ds(c * LANES, LANES)]
                    outs[b][p, pl.ds(c * LANES, LANES)] = acc
                outs[b][p, pl.ds(D, LANES)] = cont_v[g * GROUP_PAIRS + p, :]
                return c2

            lax.fori_loop(0, GROUP_PAIRS, pair, 0, unroll=2)

            pltpu.async_copy(
                outs[b],
                out_hbm.at[pl.ds(wid * PAIRS_PER_W + g * GROUP_PAIRS,
                                 GROUP_PAIRS)],
                osems[b])
        return carry

    lax.fori_loop(0, GROUPS // 2, outer, 0)

    for b in range(2):
        pltpu.make_async_copy(
            outs[b], out_hbm.at[pl.ds(0, GROUP_PAIRS)], osems[b]).wait()


@jax.jit
def _emb_sum_concat(idx_blocks, cont2d, embed_weight):
    fn = pl.kernel(
        _sc_body,
        out_type=jax.ShapeDtypeStruct((NPAIR, DOUT), jnp.float32),
        mesh=plsc.VectorSubcoreMesh(core_axis_name="c", subcore_axis_name="s"),
        compiler_params=pltpu.CompilerParams(use_tc_tiling_on_sc=False),
        scratch_types=[
            pltpu.VMEM((BLKS_PER_W, IDX_BLK), jnp.int32),
            pltpu.VMEM((PAIRS_PER_W, DC), jnp.float32),
            pltpu.VMEM((ROWS_PER_GROUP, D), jnp.float32),
            pltpu.VMEM((ROWS_PER_GROUP, D), jnp.float32),
            pltpu.VMEM((GROUP_PAIRS, DOUT), jnp.float32),
            pltpu.VMEM((GROUP_PAIRS, DOUT), jnp.float32),
            pltpu.SemaphoreType.DMA,
            pltpu.SemaphoreType.DMA,
            pltpu.SemaphoreType.DMA,
            pltpu.SemaphoreType.DMA,
        ],
    )
    return fn(idx_blocks, cont2d, embed_weight)


def kernel(ContTensor, CatTensor, LabelTensor, DoseTensor, TimeDiffTensor,
           VTensor, VancoElTensor, PtList, LengList, embed_weight):
    idx_blocks = CatTensor.astype(jnp.int32).reshape(TOTAL_BLKS, IDX_BLK)
    cont2d = ContTensor.reshape(NPAIR, DC)
    out2d = _emb_sum_concat(idx_blocks, cont2d, embed_weight)
    outEmb = out2d.reshape(B, T, DOUT)
    return (outEmb, LabelTensor, LengList, DoseTensor, TimeDiffTensor,
            VTensor, VancoElTensor, PtList)
